# baseline (speedup 1.0000x reference)
import functools

import jax
import jax.numpy as jnp
from jax import lax
from jax.experimental import pallas as pl
from jax.experimental.pallas import tpu as pltpu
from jax.experimental.pallas import tpu_sc as plsc

_HIDDEN = 64
_CHUNK = 128
_NBUF = 4


@functools.lru_cache(maxsize=None)
def _build(total, hidden):
    info = plsc.get_sparse_core_info()
    nc, ns = info.num_cores, info.num_subcores
    nw = nc * ns
    per_w = total // nw
    nchunk = per_w // _CHUNK
    assert per_w * nw == total and nchunk * _CHUNK == per_w

    mesh = plsc.VectorSubcoreMesh(core_axis_name="c", subcore_axis_name="s")

    @functools.partial(
        pl.kernel,
        mesh=mesh,
        out_type=jax.ShapeDtypeStruct((total // _CHUNK, _CHUNK, hidden), jnp.float32),
        compiler_params=pltpu.CompilerParams(use_tc_tiling_on_sc=False),
        scratch_types=[
            pltpu.VMEM((nchunk, _CHUNK), jnp.int32),
            pltpu.VMEM((_NBUF, _CHUNK, hidden), jnp.float32),
            pltpu.SemaphoreType.DMA((_NBUF,)),
        ],
    )
    def gather_k(table_hbm, idx_hbm, out_hbm, idx_v, rows_v, gsem):
        wid = lax.axis_index("s") * nc + lax.axis_index("c")
        cbase = wid * nchunk
        pltpu.sync_copy(idx_hbm.at[pl.ds(cbase, nchunk)], idx_v)

        for j in range(_NBUF):
            pltpu.async_copy(table_hbm.at[idx_v.at[j]], rows_v.at[j], gsem.at[j])

        def body(c, carry):
            p = lax.rem(c, _NBUF)
            pltpu.make_async_copy(table_hbm.at[idx_v.at[p]], rows_v.at[p], gsem.at[p]).wait()
            pltpu.sync_copy(rows_v.at[p], out_hbm.at[cbase + c])
            pltpu.async_copy(table_hbm.at[idx_v.at[c + _NBUF]], rows_v.at[p], gsem.at[p])
            return carry

        lax.fori_loop(0, nchunk - _NBUF, body, 0)

        for j in range(nchunk - _NBUF, nchunk):
            p = j % _NBUF
            pltpu.make_async_copy(table_hbm.at[idx_v.at[p]], rows_v.at[p], gsem.at[p]).wait()
            pltpu.sync_copy(rows_v.at[p], out_hbm.at[cbase + j])

    def run(embedding, idx2):
        return gather_k(embedding, idx2)

    return run


def kernel(inputs, embedding):
    b, h = inputs.shape
    total = b * h
    hidden = embedding.shape[1]
    idx2 = inputs.astype(jnp.int32).reshape(total // _CHUNK, _CHUNK)
    out = _build(total, hidden)(embedding, idx2)
    return out.reshape(b, h, hidden)


# speedup vs baseline: 1.0068x; 1.0068x over previous
import functools

import jax
import jax.numpy as jnp
from jax import lax
from jax.experimental import pallas as pl
from jax.experimental.pallas import tpu as pltpu
from jax.experimental.pallas import tpu_sc as plsc

_CHUNK = 128
_NBUF = 4
_PADW = 128


def _pad8(n):
    return (n + 7) // 8 * 8


@functools.lru_cache(maxsize=None)
def _build(total, hidden, vocab):
    info = plsc.get_sparse_core_info()
    nc, ns = info.num_cores, info.num_subcores
    nw = nc * ns
    per_w = total // nw
    nchunk = per_w // _CHUNK
    assert per_w * nw == total and nchunk * _CHUNK == per_w

    mesh = plsc.VectorSubcoreMesh(core_axis_name="c", subcore_axis_name="s")

    @functools.partial(
        pl.kernel,
        mesh=mesh,
        out_type=jax.ShapeDtypeStruct((total // _CHUNK, _CHUNK, _PADW), jnp.float32),
        compiler_params=pltpu.CompilerParams(use_tc_tiling_on_sc=True),
        scratch_types=[
            pltpu.VMEM((_pad8(nchunk), _CHUNK), jnp.int32),
            pltpu.VMEM((_NBUF, _CHUNK, _PADW), jnp.float32),
            pltpu.SemaphoreType.DMA((_NBUF,)),
        ],
    )
    def gather_k(table_hbm, idx_hbm, out_hbm, idx_v, rows_v, gsem):
        wid = lax.axis_index("s") * nc + lax.axis_index("c")
        cbase = wid * nchunk
        pltpu.sync_copy(idx_hbm.at[wid], idx_v)

        for j in range(_NBUF):
            pltpu.async_copy(table_hbm.at[idx_v.at[j]], rows_v.at[j], gsem.at[j])

        def body(c, carry):
            p = lax.rem(c, _NBUF)
            pltpu.make_async_copy(table_hbm.at[idx_v.at[p]], rows_v.at[p], gsem.at[p]).wait()
            pltpu.sync_copy(rows_v.at[p], out_hbm.at[cbase + c])
            pltpu.async_copy(table_hbm.at[idx_v.at[c + _NBUF]], rows_v.at[p], gsem.at[p])
            return carry

        lax.fori_loop(0, nchunk - _NBUF, body, 0)

        for j in range(nchunk - _NBUF, nchunk):
            p = j % _NBUF
            pltpu.make_async_copy(table_hbm.at[idx_v.at[p]], rows_v.at[p], gsem.at[p]).wait()
            pltpu.sync_copy(rows_v.at[p], out_hbm.at[cbase + j])

    return gather_k


def kernel(inputs, embedding):
    b, h = inputs.shape
    total = b * h
    vocab, hidden = embedding.shape
    nw = 32
    nchunk = total // nw // _CHUNK
    idx3 = inputs.astype(jnp.int32).reshape(nw, nchunk, _CHUNK)
    idx3 = jnp.pad(idx3, ((0, 0), (0, _pad8(nchunk) - nchunk), (0, 0)))
    table = jnp.pad(embedding, ((0, 0), (0, _PADW - hidden)))
    out = _build(total, hidden, vocab)(table, idx3)
    return out[:, :, :hidden].reshape(b, h, hidden)


# speedup vs baseline: 1.0075x; 1.0007x over previous
import functools

import jax
import jax.numpy as jnp
from jax import lax
from jax.experimental import pallas as pl
from jax.experimental.pallas import tpu as pltpu
from jax.experimental.pallas import tpu_sc as plsc

_CHUNK = 128
_NBUF = 4
_PADW = 128


def _pad8(n):
    return (n + 7) // 8 * 8


@functools.lru_cache(maxsize=None)
def _build(total, hidden, vocab):
    info = plsc.get_sparse_core_info()
    nc, ns = info.num_cores, info.num_subcores
    nw = nc * ns
    per_w = total // nw
    nchunk = per_w // _CHUNK
    assert per_w * nw == total and nchunk * _CHUNK == per_w

    mesh = plsc.VectorSubcoreMesh(core_axis_name="c", subcore_axis_name="s")

    @functools.partial(
        pl.kernel,
        mesh=mesh,
        out_type=jax.ShapeDtypeStruct((total // _CHUNK, _CHUNK, _PADW), jnp.float32),
        compiler_params=pltpu.CompilerParams(use_tc_tiling_on_sc=False),
        scratch_types=[
            pltpu.VMEM((nchunk, _CHUNK), jnp.int32),
            pltpu.VMEM((_NBUF, _CHUNK, _PADW), jnp.float32),
            pltpu.SemaphoreType.DMA((_NBUF,)),
        ],
    )
    def gather_k(table_hbm, idx_hbm, out_hbm, idx_v, rows_v, gsem):
        wid = lax.axis_index("s") * nc + lax.axis_index("c")
        cbase = wid * nchunk
        pltpu.sync_copy(idx_hbm.at[pl.ds(cbase, nchunk)], idx_v)

        for j in range(_NBUF):
            pltpu.async_copy(table_hbm.at[idx_v.at[j]], rows_v.at[j], gsem.at[j])

        def body(c, carry):
            p = lax.rem(c, _NBUF)
            pltpu.make_async_copy(table_hbm.at[idx_v.at[p]], rows_v.at[p], gsem.at[p]).wait()
            pltpu.sync_copy(rows_v.at[p], out_hbm.at[cbase + c])
            pltpu.async_copy(table_hbm.at[idx_v.at[c + _NBUF]], rows_v.at[p], gsem.at[p])
            return carry

        lax.fori_loop(0, nchunk - _NBUF, body, 0)

        for j in range(nchunk - _NBUF, nchunk):
            p = j % _NBUF
            pltpu.make_async_copy(table_hbm.at[idx_v.at[p]], rows_v.at[p], gsem.at[p]).wait()
            pltpu.sync_copy(rows_v.at[p], out_hbm.at[cbase + j])

    return gather_k


def kernel(inputs, embedding):
    b, h = inputs.shape
    total = b * h
    vocab, hidden = embedding.shape
    idx2 = inputs.astype(jnp.int32).reshape(total // _CHUNK, _CHUNK)
    table = jnp.pad(embedding, ((0, 0), (0, _PADW - hidden)))
    out = _build(total, hidden, vocab)(table, idx2)
    return out[:, :, :hidden].reshape(b, h, hidden)
